# Initial kernel scaffold; baseline (speedup 1.0000x reference)
import jax
import jax.numpy as jnp
from jax.experimental import pallas as pl


def kernel(tokens, word_embd, fc1_w, fc1_b, fc2_w, fc2_b):
    raise NotImplementedError("write your pallas kernel here")


# speedup vs baseline: 2.1267x; 2.1267x over previous
import functools

import jax
import jax.numpy as jnp
from jax import lax
from jax.experimental import pallas as pl
from jax.experimental.pallas import tpu as pltpu
from jax.experimental.pallas import tpu_sc as plsc

NC, NS = 2, 16
NW = NC * NS


def _mm_bias(x_ref, w_ref, b_ref, o_ref):
    o_ref[...] = jnp.dot(
        x_ref[...], w_ref[...], preferred_element_type=jnp.float32) + b_ref[...]


def _relu_mm_bias(x_ref, w_ref, b_ref, o_ref):
    o_ref[...] = jnp.dot(
        jax.nn.relu(x_ref[...]), w_ref[...],
        preferred_element_type=jnp.float32) + b_ref[...]


def _matmul(x, w, b, bm, body):
    m, k = x.shape
    n = w.shape[1]
    return pl.pallas_call(
        body,
        grid=(m // bm,),
        in_specs=[
            pl.BlockSpec((bm, k), lambda i: (i, 0)),
            pl.BlockSpec((k, n), lambda i: (0, 0)),
            pl.BlockSpec((1, n), lambda i: (0, 0)),
        ],
        out_specs=pl.BlockSpec((bm, n), lambda i: (i, 0)),
        out_shape=jax.ShapeDtypeStruct((m, n), jnp.float32),
    )(x, w, b.reshape(1, n))


def _gather_max(proj, tok):
    V, D = proj.shape
    SENT, W = tok.shape
    sent_per_w = SENT // NW
    CH = min(32, sent_per_w)
    n_chunks = sent_per_w // CH
    mesh = plsc.VectorSubcoreMesh(core_axis_name="c", subcore_axis_name="s")

    @functools.partial(
        pl.kernel,
        out_type=jax.ShapeDtypeStruct((SENT, D), jnp.float32),
        mesh=mesh,
        scratch_types=[
            pltpu.VMEM((CH, W), jnp.int32),
            pltpu.VMEM((W, D), jnp.float32),
            pltpu.VMEM((1, D), jnp.float32),
            pltpu.SemaphoreType.DMA,
        ],
    )
    def k(proj_hbm, tok_hbm, out_hbm, idx_v, rows_v, out_v, sem):
        wid = lax.axis_index("s") * NC + lax.axis_index("c")
        base_sent = wid * sent_per_w

        def chunk_body(ci, _):
            sent0 = base_sent + ci * CH
            pltpu.sync_copy(tok_hbm.at[pl.ds(sent0, CH)], idx_v)

            def sent_body(j, _):
                idx = idx_v[j]
                pltpu.async_copy(proj_hbm.at[idx], rows_v, sem).wait()

                def col_body(c, _):
                    o = c * 16
                    acc = rows_v[0, pl.ds(o, 16)]
                    for t in range(1, 16):
                        acc = jnp.maximum(acc, rows_v[t, pl.ds(o, 16)])
                    out_v[0, pl.ds(o, 16)] = acc
                    return 0

                lax.fori_loop(0, D // 16, col_body, 0)
                pltpu.sync_copy(out_v, out_hbm.at[pl.ds(sent0 + j, 1)])
                return 0

            lax.fori_loop(0, CH, sent_body, 0)
            return 0

        lax.fori_loop(0, n_chunks, chunk_body, 0)

    return k(proj, tok)


def kernel(tokens, word_embd, fc1_w, fc1_b, fc2_w, fc2_b):
    V, WD = word_embd.shape
    OD = fc1_w.shape[1]

    MB1 = 256
    Vp = -(-V // MB1) * MB1
    Kp = -(-WD // 128) * 128
    we = jnp.zeros((Vp, Kp), jnp.float32).at[:V, :WD].set(word_embd)
    w1 = jnp.zeros((Kp, OD), jnp.float32).at[:WD, :].set(fc1_w)

    proj = _matmul(we, w1, fc1_b, MB1, _mm_bias)
    pooled = _gather_max(proj, tokens.astype(jnp.int32))
    return _matmul(pooled, fc2_w, fc2_b, 512, _relu_mm_bias)


# speedup vs baseline: 4.5570x; 2.1427x over previous
import functools

import jax
import jax.numpy as jnp
from jax import lax
from jax.experimental import pallas as pl
from jax.experimental.pallas import tpu as pltpu
from jax.experimental.pallas import tpu_sc as plsc

NC, NS = 2, 16
NW = NC * NS


def _key(u):
    return u ^ ((u >> 15) * jnp.uint32(0x7FFF))


def _proj_body(x_ref, w_ref, b_ref, o_ref):
    acc = jnp.dot(x_ref[...].astype(jnp.bfloat16), w_ref[...],
                  preferred_element_type=jnp.float32) + b_ref[...]
    bf = acc.astype(jnp.bfloat16)
    n = bf.shape[1]
    u = lax.bitcast_convert_type(bf, jnp.uint16).astype(jnp.uint32)
    k = _key(u)
    word = k[:, :n // 2] | (k[:, n // 2:] << 16)
    o_ref[...] = lax.bitcast_convert_type(word, jnp.int32)


def _proj_matmul(x, w, b, bm):
    m, k = x.shape
    n = w.shape[1]
    return pl.pallas_call(
        _proj_body,
        grid=(pl.cdiv(m, bm),),
        in_specs=[
            pl.BlockSpec((bm, k), lambda i: (i, 0)),
            pl.BlockSpec((k, n), lambda i: (0, 0)),
            pl.BlockSpec((1, n), lambda i: (0, 0)),
        ],
        out_specs=pl.BlockSpec((bm, n // 2), lambda i: (i, 0)),
        out_shape=jax.ShapeDtypeStruct((m, n // 2), jnp.int32),
    )(x, w.astype(jnp.bfloat16), b.reshape(1, n))


def _fc2_body(x_ref, w_ref, b_ref, o_ref):
    xu = lax.bitcast_convert_type(x_ref[...], jnp.uint32)
    lo = _key(xu & jnp.uint32(0xFFFF))
    hi = _key(xu >> 16)
    lo_bf = lax.bitcast_convert_type(lo.astype(jnp.uint16), jnp.bfloat16)
    hi_bf = lax.bitcast_convert_type(hi.astype(jnp.uint16), jnp.bfloat16)
    xb = jnp.concatenate([lo_bf, hi_bf], axis=1)
    o_ref[...] = jnp.dot(jax.nn.relu(xb), w_ref[...],
                         preferred_element_type=jnp.float32) + b_ref[...]


def _fc2_matmul(x, w, b, bm):
    m, kw = x.shape
    k, n = w.shape
    return pl.pallas_call(
        _fc2_body,
        grid=(m // bm,),
        in_specs=[
            pl.BlockSpec((bm, kw), lambda i: (i, 0)),
            pl.BlockSpec((k, n), lambda i: (0, 0)),
            pl.BlockSpec((1, n), lambda i: (0, 0)),
        ],
        out_specs=pl.BlockSpec((bm, n), lambda i: (i, 0)),
        out_shape=jax.ShapeDtypeStruct((m, n), jnp.float32),
    )(x, w.astype(jnp.bfloat16), b.reshape(1, n))


def _gather_max(proj, tok_flat):
    V, Dw = proj.shape
    W = 16
    SENT = tok_flat.shape[0] // W
    sent_per_w = SENT // NW
    BODY = 4
    n_bodies = sent_per_w // BODY
    mesh = plsc.VectorSubcoreMesh(core_axis_name="c", subcore_axis_name="s")

    @functools.partial(
        pl.kernel,
        out_type=jax.ShapeDtypeStruct((SENT, Dw), jnp.int32),
        mesh=mesh,
        scratch_types=[
            pltpu.VMEM((sent_per_w * W,), jnp.int32),
            pltpu.VMEM((2 * W, Dw), jnp.int32),
            pltpu.VMEM((2 * W, Dw), jnp.int32),
            pltpu.VMEM((BODY, Dw), jnp.int32),
            pltpu.SemaphoreType.DMA,
            pltpu.SemaphoreType.DMA,
        ],
    )
    def k(proj_hbm, tok_hbm, out_hbm, idx_v, buf_a, buf_b, out_v, sem_a, sem_b):
        wid = lax.axis_index("s") * NC + lax.axis_index("c")
        base = wid * sent_per_w
        pltpu.sync_copy(tok_hbm.at[pl.ds(base * W, sent_per_w * W)], idx_v)

        def fire(local_s, buf, sem):
            for h in range(2):
                idx = idx_v[pl.ds((local_s + h) * W, W)]
                pltpu.async_copy(
                    proj_hbm.at[idx], buf.at[pl.ds(h * W, W)], sem)

        def drain(buf, sem):
            for h in range(2):
                pltpu.make_async_copy(
                    proj_hbm.at[pl.ds(0, W)], buf.at[pl.ds(h * W, W)], sem
                ).wait()

        def reduce_pair(buf, o0):
            def col_body(c, _):
                o = c * 16
                for h in range(2):
                    w0 = buf[h * W, pl.ds(o, 16)]
                    m_hi = w0
                    m_lo = w0 << 16
                    for t in range(1, W):
                        wt = buf[h * W + t, pl.ds(o, 16)]
                        m_hi = jnp.maximum(m_hi, wt)
                        m_lo = jnp.maximum(m_lo, wt << 16)
                    lo16 = (m_lo.astype(jnp.uint32) >> 16).astype(jnp.int32)
                    pooled = (m_hi & jnp.int32(-65536)) | lo16
                    out_v[o0 + h, pl.ds(o, 16)] = pooled
                return 0

            lax.fori_loop(0, Dw // 16, col_body, 0)

        fire(0, buf_a, sem_a)

        def body(q, _):
            s0 = q * BODY
            fire(s0 + 2, buf_b, sem_b)
            drain(buf_a, sem_a)
            reduce_pair(buf_a, 0)

            @pl.when(q < n_bodies - 1)
            def _():
                fire(s0 + 4, buf_a, sem_a)

            drain(buf_b, sem_b)
            reduce_pair(buf_b, 2)
            pltpu.sync_copy(out_v, out_hbm.at[pl.ds(base + s0, BODY)])
            return 0

        lax.fori_loop(0, n_bodies, body, 0)

    return k(proj, tok_flat)


def kernel(tokens, word_embd, fc1_w, fc1_b, fc2_w, fc2_b):
    proj = _proj_matmul(word_embd, fc1_w, fc1_b, 256)
    tok_flat = tokens.astype(jnp.int32).reshape(-1)
    pooled = _gather_max(proj, tok_flat)
    return _fc2_matmul(pooled, fc2_w, fc2_b, 512)


# speedup vs baseline: 4.5907x; 1.0074x over previous
import functools

import jax
import jax.numpy as jnp
from jax import lax
from jax.experimental import pallas as pl
from jax.experimental.pallas import tpu as pltpu
from jax.experimental.pallas import tpu_sc as plsc

NC, NS = 2, 16
NW = NC * NS


def _proj_body(x_ref, w_ref, b_ref, o_ref):
    acc = jnp.dot(x_ref[...].astype(jnp.bfloat16), w_ref[...],
                  preferred_element_type=jnp.float32) + b_ref[...]
    bf = jax.nn.relu(acc).astype(jnp.bfloat16)
    n = bf.shape[1]
    u = lax.bitcast_convert_type(bf, jnp.uint16).astype(jnp.uint32)
    word = u[:, :n // 2] | (u[:, n // 2:] << 16)
    o_ref[...] = lax.bitcast_convert_type(word, jnp.int32)


def _proj_matmul(x, w, b, bm):
    m, k = x.shape
    n = w.shape[1]
    return pl.pallas_call(
        _proj_body,
        grid=(pl.cdiv(m, bm),),
        in_specs=[
            pl.BlockSpec((bm, k), lambda i: (i, 0)),
            pl.BlockSpec((k, n), lambda i: (0, 0)),
            pl.BlockSpec((1, n), lambda i: (0, 0)),
        ],
        out_specs=pl.BlockSpec((bm, n // 2), lambda i: (i, 0)),
        out_shape=jax.ShapeDtypeStruct((m, n // 2), jnp.int32),
    )(x, w.astype(jnp.bfloat16), b.reshape(1, n))


def _fc2_body(x_ref, w_ref, b_ref, o_ref):
    xu = lax.bitcast_convert_type(x_ref[...], jnp.uint32)
    lo = (xu & jnp.uint32(0xFFFF)).astype(jnp.uint16)
    hi = (xu >> 16).astype(jnp.uint16)
    xb = jnp.concatenate(
        [lax.bitcast_convert_type(lo, jnp.bfloat16),
         lax.bitcast_convert_type(hi, jnp.bfloat16)], axis=1)
    o_ref[...] = jnp.dot(xb, w_ref[...],
                         preferred_element_type=jnp.float32) + b_ref[...]


def _fc2_matmul(x, w, b, bm):
    m, kw = x.shape
    k, n = w.shape
    return pl.pallas_call(
        _fc2_body,
        grid=(m // bm,),
        in_specs=[
            pl.BlockSpec((bm, kw), lambda i: (i, 0)),
            pl.BlockSpec((k, n), lambda i: (0, 0)),
            pl.BlockSpec((1, n), lambda i: (0, 0)),
        ],
        out_specs=pl.BlockSpec((bm, n), lambda i: (i, 0)),
        out_shape=jax.ShapeDtypeStruct((m, n), jnp.float32),
    )(x, w.astype(jnp.bfloat16), b.reshape(1, n))


def _gather_max(proj, tok_flat):
    V, Dw = proj.shape
    W = 16
    SENT = tok_flat.shape[0] // W
    sent_per_w = SENT // NW
    NBUF = 4
    BODY = 8
    n_bodies = sent_per_w // BODY
    mesh = plsc.VectorSubcoreMesh(core_axis_name="c", subcore_axis_name="s")

    @functools.partial(
        pl.kernel,
        out_type=jax.ShapeDtypeStruct((SENT, Dw), jnp.int32),
        mesh=mesh,
        scratch_types=[
            pltpu.VMEM((sent_per_w * W,), jnp.int32),
            [pltpu.VMEM((W, Dw), jnp.int32) for _ in range(NBUF)],
            pltpu.VMEM((BODY, Dw), jnp.int32),
            [pltpu.SemaphoreType.DMA for _ in range(NBUF)],
        ],
    )
    def k(proj_hbm, tok_hbm, out_hbm, idx_v, bufs, out_v, sems):
        wid = lax.axis_index("s") * NC + lax.axis_index("c")
        base = wid * sent_per_w
        pltpu.sync_copy(tok_hbm.at[pl.ds(base * W, sent_per_w * W)], idx_v)

        def fire(local_s, b):
            idx = idx_v[pl.ds(local_s * W, W)]
            pltpu.async_copy(proj_hbm.at[idx], bufs[b], sems[b])

        def drain(b):
            pltpu.make_async_copy(
                proj_hbm.at[pl.ds(0, W)], bufs[b], sems[b]).wait()

        def reduce_sent(b, o_row):
            buf = bufs[b]

            def col_body(c, _):
                for half in range(2):
                    o = (2 * c + half) * 16
                    w0 = buf[0, pl.ds(o, 16)]
                    m_hi = w0
                    m_lo = w0 << 16
                    for t in range(1, W):
                        wt = buf[t, pl.ds(o, 16)]
                        m_hi = jnp.maximum(m_hi, wt)
                        m_lo = jnp.maximum(m_lo, wt << 16)
                    lo16 = (m_lo.astype(jnp.uint32) >> 16).astype(jnp.int32)
                    out_v[o_row, pl.ds(o, 16)] = \
                        (m_hi & jnp.int32(-65536)) | lo16
                return 0

            lax.fori_loop(0, Dw // 32, col_body, 0)

        for p in range(NBUF - 1):
            fire(p, p)

        def body(q, _):
            s0 = q * BODY
            for u in range(BODY):
                b = u % NBUF
                drain(b)
                reduce_sent(b, u)

                @pl.when(s0 + u + NBUF - 1 < sent_per_w)
                def _():
                    fire(s0 + u + NBUF - 1, (u + NBUF - 1) % NBUF)

            pltpu.sync_copy(out_v, out_hbm.at[pl.ds(base + s0, BODY)])
            return 0

        lax.fori_loop(0, n_bodies, body, 0)

    return k(proj, tok_flat)


def kernel(tokens, word_embd, fc1_w, fc1_b, fc2_w, fc2_b):
    proj = _proj_matmul(word_embd, fc1_w, fc1_b, 256)
    tok_flat = tokens.astype(jnp.int32).reshape(-1)
    pooled = _gather_max(proj, tok_flat)
    return _fc2_matmul(pooled, fc2_w, fc2_b, 512)


# speedup vs baseline: 6.0820x; 1.3248x over previous
import functools

import jax
import jax.numpy as jnp
from jax import lax
from jax.experimental import pallas as pl
from jax.experimental.pallas import tpu as pltpu
from jax.experimental.pallas import tpu_sc as plsc

NC, NS = 2, 16
NW = NC * NS


def _proj_body(x_ref, w_ref, b_ref, o_ref):
    acc = jnp.dot(x_ref[...].astype(jnp.bfloat16), w_ref[...],
                  preferred_element_type=jnp.float32) + b_ref[...]
    bf = jax.nn.relu(acc).astype(jnp.bfloat16)
    n = bf.shape[1]
    u = lax.bitcast_convert_type(bf, jnp.uint16).astype(jnp.uint32)
    word = u[:, :n // 2] | (u[:, n // 2:] << 16)
    o_ref[...] = lax.bitcast_convert_type(word, jnp.int32)


def _proj_matmul(x, w, b, bm):
    m, k = x.shape
    n = w.shape[1]
    return pl.pallas_call(
        _proj_body,
        grid=(pl.cdiv(m, bm),),
        in_specs=[
            pl.BlockSpec((bm, k), lambda i: (i, 0)),
            pl.BlockSpec((k, n), lambda i: (0, 0)),
            pl.BlockSpec((1, n), lambda i: (0, 0)),
        ],
        out_specs=pl.BlockSpec((bm, n // 2), lambda i: (i, 0)),
        out_shape=jax.ShapeDtypeStruct((m, n // 2), jnp.int32),
    )(x, w.astype(jnp.bfloat16), b.reshape(1, n))


def _fc2_body(x_ref, w_ref, b_ref, o_ref):
    xu = lax.bitcast_convert_type(x_ref[...], jnp.uint32)
    lo = (xu & jnp.uint32(0xFFFF)).astype(jnp.uint16)
    hi = (xu >> 16).astype(jnp.uint16)
    xb = jnp.concatenate(
        [lax.bitcast_convert_type(lo, jnp.bfloat16),
         lax.bitcast_convert_type(hi, jnp.bfloat16)], axis=1)
    o_ref[...] = jnp.dot(xb, w_ref[...],
                         preferred_element_type=jnp.float32) + b_ref[...]


def _fc2_matmul(x, w, b, bm):
    m, kw = x.shape
    k, n = w.shape
    return pl.pallas_call(
        _fc2_body,
        grid=(m // bm,),
        in_specs=[
            pl.BlockSpec((bm, kw), lambda i: (i, 0)),
            pl.BlockSpec((k, n), lambda i: (0, 0)),
            pl.BlockSpec((1, n), lambda i: (0, 0)),
        ],
        out_specs=pl.BlockSpec((bm, n), lambda i: (i, 0)),
        out_shape=jax.ShapeDtypeStruct((m, n), jnp.float32),
    )(x, w.astype(jnp.bfloat16), b.reshape(1, n))


def _gather_max(proj, tok_flat):
    V, Dw = proj.shape
    W = 16
    SENT = tok_flat.shape[0] // W
    sent_per_w = SENT // NW
    NBUF = 4
    BODY = 8
    n_bodies = sent_per_w // BODY
    mesh = plsc.VectorSubcoreMesh(core_axis_name="c", subcore_axis_name="s")

    @functools.partial(
        pl.kernel,
        out_type=jax.ShapeDtypeStruct((SENT, Dw), jnp.int32),
        mesh=mesh,
        scratch_types=[
            pltpu.VMEM((sent_per_w * W,), jnp.int32),
            [pltpu.VMEM((W, Dw), jnp.int32) for _ in range(NBUF)],
            pltpu.VMEM((BODY, Dw), jnp.int32),
            [pltpu.SemaphoreType.DMA for _ in range(NBUF)],
        ],
    )
    def k(proj_hbm, tok_hbm, out_hbm, idx_v, bufs, out_v, sems):
        wid = lax.axis_index("s") * NC + lax.axis_index("c")
        base = wid * sent_per_w
        pltpu.sync_copy(tok_hbm.at[pl.ds(base * W, sent_per_w * W)], idx_v)

        def fire(local_s, b):
            idx = idx_v[pl.ds(local_s * W, W)]
            pltpu.async_copy(proj_hbm.at[idx], bufs[b], sems[b])

        def drain(b):
            pltpu.make_async_copy(
                proj_hbm.at[pl.ds(0, W)], bufs[b], sems[b]).wait()

        def reduce_sent(b, o_row):
            buf = bufs[b]

            def col_body(c, _):
                for half in range(2):
                    o = (2 * c + half) * 16
                    w0 = buf[0, pl.ds(o, 16)].astype(jnp.uint32)
                    m_hi = w0
                    m_lo = w0 & jnp.uint32(0xFFFF)
                    for t in range(1, W):
                        wt = buf[t, pl.ds(o, 16)].astype(jnp.uint32)
                        m_hi = jnp.maximum(m_hi, wt)
                        m_lo = jnp.maximum(m_lo, wt & jnp.uint32(0xFFFF))
                    out_v[o_row, pl.ds(o, 16)] = (
                        (m_hi & jnp.uint32(0xFFFF0000)) | m_lo
                    ).astype(jnp.int32)
                return 0

            lax.fori_loop(0, Dw // 32, col_body, 0)

        for p in range(NBUF - 1):
            fire(p, p)

        def body(q, _):
            s0 = q * BODY
            for u in range(BODY):
                b = u % NBUF
                drain(b)
                reduce_sent(b, u)

                @pl.when(s0 + u + NBUF - 1 < sent_per_w)
                def _():
                    fire(s0 + u + NBUF - 1, (u + NBUF - 1) % NBUF)

            pltpu.sync_copy(out_v, out_hbm.at[pl.ds(base + s0, BODY)])
            return 0

        lax.fori_loop(0, n_bodies, body, 0)

    return k(proj, tok_flat)


def kernel(tokens, word_embd, fc1_w, fc1_b, fc2_w, fc2_b):
    proj = _proj_matmul(word_embd, fc1_w, fc1_b, 256)
    tok_flat = tokens.astype(jnp.int32).reshape(-1)
    pooled = _gather_max(proj, tok_flat)
    return _fc2_matmul(pooled, fc2_w, fc2_b, 512)


# speedup vs baseline: 6.2872x; 1.0337x over previous
import functools

import jax
import jax.numpy as jnp
from jax import lax
from jax.experimental import pallas as pl
from jax.experimental.pallas import tpu as pltpu
from jax.experimental.pallas import tpu_sc as plsc

NC, NS = 2, 16
NW = NC * NS


def _proj_body(x_ref, w_ref, b_ref, o_ref):
    acc = jnp.dot(x_ref[...].astype(jnp.bfloat16), w_ref[...],
                  preferred_element_type=jnp.float32) + b_ref[...]
    bf = jax.nn.relu(acc).astype(jnp.bfloat16)
    n = bf.shape[1]
    u = lax.bitcast_convert_type(bf, jnp.uint16).astype(jnp.uint32)
    word = u[:, :n // 2] | (u[:, n // 2:] << 16)
    o_ref[...] = lax.bitcast_convert_type(word, jnp.int32)


def _proj_matmul(x, w, b, bm):
    m, k = x.shape
    n = w.shape[1]
    return pl.pallas_call(
        _proj_body,
        grid=(pl.cdiv(m, bm),),
        in_specs=[
            pl.BlockSpec((bm, k), lambda i: (i, 0)),
            pl.BlockSpec((k, n), lambda i: (0, 0)),
            pl.BlockSpec((1, n), lambda i: (0, 0)),
        ],
        out_specs=pl.BlockSpec((bm, n // 2), lambda i: (i, 0)),
        out_shape=jax.ShapeDtypeStruct((m, n // 2), jnp.int32),
    )(x, w.astype(jnp.bfloat16), b.reshape(1, n))


def _fc2_body(x_ref, w_ref, b_ref, o_ref):
    xu = lax.bitcast_convert_type(x_ref[...], jnp.uint32)
    lo = (xu & jnp.uint32(0xFFFF)).astype(jnp.uint16)
    hi = (xu >> 16).astype(jnp.uint16)
    xb = jnp.concatenate(
        [lax.bitcast_convert_type(lo, jnp.bfloat16),
         lax.bitcast_convert_type(hi, jnp.bfloat16)], axis=1)
    o_ref[...] = jnp.dot(xb, w_ref[...],
                         preferred_element_type=jnp.float32) + b_ref[...]


def _fc2_matmul(x, w, b, bm):
    m, kw = x.shape
    k, n = w.shape
    return pl.pallas_call(
        _fc2_body,
        grid=(m // bm,),
        in_specs=[
            pl.BlockSpec((bm, kw), lambda i: (i, 0)),
            pl.BlockSpec((k, n), lambda i: (0, 0)),
            pl.BlockSpec((1, n), lambda i: (0, 0)),
        ],
        out_specs=pl.BlockSpec((bm, n), lambda i: (i, 0)),
        out_shape=jax.ShapeDtypeStruct((m, n), jnp.float32),
    )(x, w.astype(jnp.bfloat16), b.reshape(1, n))


def _gather_max(proj, tok_flat):
    V, Dw = proj.shape
    W = 16
    SENT = tok_flat.shape[0] // W
    sent_per_w = SENT // NW
    NBUF = 4
    BODY = 8
    n_bodies = sent_per_w // BODY
    mesh = plsc.VectorSubcoreMesh(core_axis_name="c", subcore_axis_name="s")

    @functools.partial(
        pl.kernel,
        out_type=jax.ShapeDtypeStruct((SENT, Dw), jnp.int32),
        mesh=mesh,
        scratch_types=[
            pltpu.VMEM((sent_per_w * W,), jnp.int32),
            [pltpu.VMEM((W, Dw), jnp.int32) for _ in range(NBUF)],
            pltpu.VMEM((BODY, Dw), jnp.int32),
            [pltpu.SemaphoreType.DMA for _ in range(NBUF)],
        ],
    )
    def k(proj_hbm, tok_hbm, out_hbm, idx_v, bufs, out_v, sems):
        wid = lax.axis_index("s") * NC + lax.axis_index("c")
        base = wid * sent_per_w
        pltpu.sync_copy(tok_hbm.at[pl.ds(base * W, sent_per_w * W)], idx_v)

        def fire(local_s, b):
            idx = idx_v[pl.ds(local_s * W, W)]
            pltpu.async_copy(proj_hbm.at[idx], bufs[b], sems[b])

        def drain(b):
            pltpu.make_async_copy(
                proj_hbm.at[pl.ds(0, W)], bufs[b], sems[b]).wait()

        def reduce_sent(b, o_row):
            buf = bufs[b]

            @plsc.parallel_loop(0, Dw // 32, 1, unroll=2)
            def col_body(c):
                for half in range(2):
                    o = (2 * c + half) * 16
                    w0 = buf[0, pl.ds(o, 16)].astype(jnp.uint32)
                    m_hi = w0
                    m_lo = w0 & jnp.uint32(0xFFFF)
                    for t in range(1, W):
                        wt = buf[t, pl.ds(o, 16)].astype(jnp.uint32)
                        m_hi = jnp.maximum(m_hi, wt)
                        m_lo = jnp.maximum(m_lo, wt & jnp.uint32(0xFFFF))
                    out_v[o_row, pl.ds(o, 16)] = (
                        (m_hi & jnp.uint32(0xFFFF0000)) | m_lo
                    ).astype(jnp.int32)

        for p in range(NBUF - 1):
            fire(p, p)

        def body(q, _):
            s0 = q * BODY
            for u in range(BODY):
                b = u % NBUF
                drain(b)
                reduce_sent(b, u)

                @pl.when(s0 + u + NBUF - 1 < sent_per_w)
                def _():
                    fire(s0 + u + NBUF - 1, (u + NBUF - 1) % NBUF)

            pltpu.sync_copy(out_v, out_hbm.at[pl.ds(base + s0, BODY)])
            return 0

        lax.fori_loop(0, n_bodies, body, 0)

    return k(proj, tok_flat)


def kernel(tokens, word_embd, fc1_w, fc1_b, fc2_w, fc2_b):
    proj = _proj_matmul(word_embd, fc1_w, fc1_b, 256)
    tok_flat = tokens.astype(jnp.int32).reshape(-1)
    pooled = _gather_max(proj, tok_flat)
    return _fc2_matmul(pooled, fc2_w, fc2_b, 512)
